# baseline (speedup 1.0000x reference)
import jax
import jax.numpy as jnp
from jax import lax
from jax.experimental import pallas as pl
from jax.experimental.pallas import tpu as pltpu
from jax.experimental.pallas import tpu_sc as plsc

E = 64
TOPK = 1
D_MODEL = 768
D_MLP = 1024
CAP = 256
N_TOK = 4096
NC, NS = 2, 16
NW = NC * NS
TPW = N_TOK // NW
DUMMY = E * CAP

def _mesh():
    return plsc.VectorSubcoreMesh(core_axis_name="c", subcore_axis_name="s")


def _sc_params():
    return pltpu.CompilerParams(needs_layout_passes=False)


def _wid():
    return lax.axis_index("s") * NC + lax.axis_index("c")


def _run_hist(eid):
    @pl.kernel(
        out_type=jax.ShapeDtypeStruct((NW, E), jnp.int32),
        mesh=_mesh(),
        compiler_params=_sc_params(),
        scratch_types=[
            pltpu.VMEM((TPW,), jnp.int32),
            pltpu.VMEM((E,), jnp.int32),
        ],
    )
    def k(eid_hbm, hist_hbm, eid_v, hist_v):
        w = _wid()
        iota = lax.iota(jnp.int32, 16)
        pltpu.sync_copy(eid_hbm.at[pl.ds(w * TPW, TPW)], eid_v)
        for c in range(E // 16):
            hist_v[pl.ds(c * 16, 16)] = jnp.zeros((16,), jnp.int32)

        @pl.loop(0, TPW // 16)
        def _(c):
            ev = eid_v[pl.ds(c * 16, 16)]
            for lane in range(16):
                e = ev[lane]
                hb = (e // 16) * 16
                hv = hist_v[pl.ds(hb, 16)]
                lm = iota == (e - hb)
                hist_v[pl.ds(hb, 16)] = jnp.where(lm, hv + 1, hv)

        pltpu.sync_copy(hist_v, hist_hbm.at[w])

    return k(eid)


_HR = TPW // 2


def _run_dispatch(eid, hist, xf):
    @pl.kernel(
        out_type=(
            jax.ShapeDtypeStruct((NW, 2, _HR), jnp.int32),
            jax.ShapeDtypeStruct((DUMMY + 8, D_MODEL), jnp.float32),
            jax.ShapeDtypeStruct((E,), jnp.int32),
        ),
        mesh=_mesh(),
        compiler_params=_sc_params(),
        scratch_types=[
            pltpu.VMEM((TPW,), jnp.int32),
            pltpu.VMEM((NW, E), jnp.int32),
            pltpu.VMEM((E,), jnp.int32),
            pltpu.VMEM((E,), jnp.int32),
            pltpu.VMEM((2, _HR), jnp.int32),
            pltpu.VMEM((2, _HR, D_MODEL), jnp.float32),
            pltpu.SemaphoreType.DMA,
            pltpu.SemaphoreType.DMA,
            pltpu.SemaphoreType.DMA,
            pltpu.SemaphoreType.DMA,
        ],
    )
    def k(eid_hbm, hist_hbm, xf_hbm, gidx_hbm, xs_hbm, counts_hbm,
          eid_v, allhist_v, base_v, counts_v, gidx_v, rows_v,
          sem_a, sem_b, sem_c, sem_d):
        w = _wid()
        tok0 = w * TPW
        in0 = pltpu.async_copy(xf_hbm.at[pl.ds(tok0, _HR)],
                               rows_v.at[0], sem_a)
        in1 = pltpu.async_copy(xf_hbm.at[pl.ds(tok0 + _HR, _HR)],
                               rows_v.at[1], sem_b)
        pltpu.sync_copy(eid_hbm.at[pl.ds(tok0, TPW)], eid_v)
        pltpu.sync_copy(hist_hbm, allhist_v)
        for c in range(E // 16):
            base_v[pl.ds(c * 16, 16)] = jnp.zeros((16,), jnp.int32)

        def acc(wp, carry):
            for c in range(E // 16):
                sl = pl.ds(c * 16, 16)
                base_v[sl] = base_v[sl] + allhist_v[wp, sl]
            return carry

        lax.fori_loop(0, w, acc, 0)

        @pl.when(w == 0)
        def _():
            for c in range(E // 16):
                sl = pl.ds(c * 16, 16)
                tot = jnp.zeros((16,), jnp.int32)
                for wp in range(NW):
                    tot = tot + allhist_v[wp, sl]
                counts_v[sl] = jnp.minimum(tot, CAP)
            pltpu.sync_copy(counts_v, counts_hbm)

        iota = lax.iota(jnp.int32, 16)

        @pl.loop(0, TPW // 16)
        def _(c):
            ev = eid_v[pl.ds(c * 16, 16)]
            gv = jnp.zeros((16,), jnp.int32)
            for lane in range(16):
                e = ev[lane]
                hb = (e // 16) * 16
                bv = base_v[pl.ds(hb, 16)]
                lm = iota == (e - hb)
                r = jnp.max(jnp.where(lm, bv, -1))
                base_v[pl.ds(hb, 16)] = jnp.where(lm, bv + 1, bv)
                g = jnp.where(r < CAP, e * CAP + r, DUMMY)
                gv = jnp.where(iota == lane, g, gv)
            h = c // (_HR // 16)
            col = (c % (_HR // 16)) * 16
            gidx_v[h, pl.ds(col, 16)] = gv

        pltpu.sync_copy(gidx_v, gidx_hbm.at[w])
        in0.wait()
        out0 = pltpu.async_copy(rows_v.at[0], xs_hbm.at[gidx_v.at[0]], sem_c)
        in1.wait()
        out1 = pltpu.async_copy(rows_v.at[1], xs_hbm.at[gidx_v.at[1]], sem_d)
        out0.wait()
        out1.wait()

    return k(eid, hist, xf)


_BM = 64
_MB = CAP // _BM


def _run_mlp(counts, xs, W1, W2, W3):
    def body(counts_ref, xs_ref, w1_ref, w3_ref, w2_ref, out_ref):
        e = pl.program_id(0)

        @pl.when(e == E)
        def _():
            out_ref[...] = jnp.zeros(out_ref.shape, out_ref.dtype)

        @pl.when(e < E)
        def _():
            cur = xs_ref[...]
            curb = cur.astype(jnp.bfloat16)
            dn = (((1,), (1,)), ((), ()))
            h1 = lax.dot_general(curb, w1_ref[0].astype(jnp.bfloat16), dn,
                                 preferred_element_type=jnp.float32)
            h3 = lax.dot_general(curb, w3_ref[0].astype(jnp.bfloat16), dn,
                                 preferred_element_type=jnp.float32)
            hh = h1 * jax.nn.sigmoid(h1) * h3
            o = lax.dot_general(hh.astype(jnp.bfloat16),
                                w2_ref[0].astype(jnp.bfloat16), dn,
                                preferred_element_type=jnp.float32)
            out_ref[0] = o * cur[:, 0:1]

    return pl.pallas_call(
        body,
        grid=(E + 1,),
        in_specs=[
            pl.BlockSpec(memory_space=pltpu.SMEM),
            pl.BlockSpec((CAP, D_MODEL), lambda e: (jnp.minimum(e, E - 1), 0)),
            pl.BlockSpec((1, D_MLP, D_MODEL), lambda e: (jnp.minimum(e, E - 1), 0, 0)),
            pl.BlockSpec((1, D_MLP, D_MODEL), lambda e: (jnp.minimum(e, E - 1), 0, 0)),
            pl.BlockSpec((1, D_MODEL, D_MLP), lambda e: (jnp.minimum(e, E - 1), 0, 0)),
        ],
        out_specs=pl.BlockSpec((1, CAP, D_MODEL), lambda e: (e, 0, 0)),
        out_shape=jax.ShapeDtypeStruct((E + 1, CAP, D_MODEL), jnp.float32),
    )(counts, xs, W1, W3, W2)


def _run_combine(gidx, ys_flat):
    @pl.kernel(
        out_type=jax.ShapeDtypeStruct((N_TOK, D_MODEL), jnp.float32),
        mesh=_mesh(),
        compiler_params=_sc_params(),
        scratch_types=[
            pltpu.VMEM((2, _HR), jnp.int32),
            pltpu.VMEM((2, _HR, D_MODEL), jnp.float32),
            pltpu.SemaphoreType.DMA,
            pltpu.SemaphoreType.DMA,
            pltpu.SemaphoreType.DMA,
            pltpu.SemaphoreType.DMA,
        ],
    )
    def k(gidx_hbm, ys_hbm, out_hbm, idx_v, rows_v, sem_a, sem_b, sem_c, sem_d):
        w = _wid()
        tok0 = w * TPW
        pltpu.sync_copy(gidx_hbm.at[w], idx_v)
        g0 = pltpu.async_copy(ys_hbm.at[idx_v.at[0]], rows_v.at[0], sem_a)
        g1 = pltpu.async_copy(ys_hbm.at[idx_v.at[1]], rows_v.at[1], sem_b)
        g0.wait()
        o0 = pltpu.async_copy(rows_v.at[0], out_hbm.at[pl.ds(tok0, _HR)], sem_c)
        g1.wait()
        o1 = pltpu.async_copy(rows_v.at[1], out_hbm.at[pl.ds(tok0 + _HR, _HR)],
                              sem_d)
        o0.wait()
        o1.wait()

    return k(gidx, ys_flat)


def kernel(x, W_gate, W1, W2, W3):
    batch, pos, d_model = x.shape
    xf = x.reshape(-1, d_model)
    gate_logits = xf @ W_gate.T
    weights = jax.nn.softmax(gate_logits.astype(jnp.float32), axis=1)
    _, expert_indices = jax.lax.top_k(weights, TOPK)
    eid = expert_indices[:, 0].astype(jnp.int32)

    hist = _run_hist(eid)
    gidx, xs, counts = _run_dispatch(eid, hist, xf)
    ys = _run_mlp(counts, xs, W1, W2, W3)
    out = _run_combine(gidx, ys.reshape((E + 1) * CAP, D_MODEL))
    return out.reshape(batch, pos, d_model)


# speedup vs baseline: 1.0787x; 1.0787x over previous
import jax
import jax.numpy as jnp
from jax import lax
from jax.experimental import pallas as pl
from jax.experimental.pallas import tpu as pltpu
from jax.experimental.pallas import tpu_sc as plsc

E = 64
TOPK = 1
D_MODEL = 768
D_MLP = 1024
CAP = 256
N_TOK = 4096
NC, NS = 2, 16
NW = NC * NS
TPW = N_TOK // NW
DUMMY = E * CAP

def _mesh():
    return plsc.VectorSubcoreMesh(core_axis_name="c", subcore_axis_name="s")


def _sc_params():
    return pltpu.CompilerParams(needs_layout_passes=False)


def _wid():
    return lax.axis_index("s") * NC + lax.axis_index("c")


def _run_hist(eid):
    @pl.kernel(
        out_type=jax.ShapeDtypeStruct((NW, E), jnp.int32),
        mesh=_mesh(),
        compiler_params=_sc_params(),
        scratch_types=[
            pltpu.VMEM((TPW,), jnp.int32),
            pltpu.VMEM((E,), jnp.int32),
        ],
    )
    def k(eid_hbm, hist_hbm, eid_v, hist_v):
        w = _wid()
        iota = lax.iota(jnp.int32, 16)
        pltpu.sync_copy(eid_hbm.at[pl.ds(w * TPW, TPW)], eid_v)
        for c in range(E // 16):
            hist_v[pl.ds(c * 16, 16)] = jnp.zeros((16,), jnp.int32)

        @pl.loop(0, TPW // 16)
        def _(c):
            ev = eid_v[pl.ds(c * 16, 16)]
            for lane in range(16):
                e = ev[lane]
                hb = (e // 16) * 16
                hv = hist_v[pl.ds(hb, 16)]
                lm = iota == (e - hb)
                hist_v[pl.ds(hb, 16)] = jnp.where(lm, hv + 1, hv)

        pltpu.sync_copy(hist_v, hist_hbm.at[w])

    return k(eid)


_HR = TPW // 2


def _run_dispatch(eid, hist, xf):
    @pl.kernel(
        out_type=(
            jax.ShapeDtypeStruct((NW, 2, _HR), jnp.int32),
            jax.ShapeDtypeStruct((DUMMY + 8, D_MODEL), jnp.float32),
            jax.ShapeDtypeStruct((E,), jnp.int32),
        ),
        mesh=_mesh(),
        compiler_params=_sc_params(),
        scratch_types=[
            pltpu.VMEM((TPW,), jnp.int32),
            pltpu.VMEM((NW, E), jnp.int32),
            pltpu.VMEM((E,), jnp.int32),
            pltpu.VMEM((E,), jnp.int32),
            pltpu.VMEM((2, _HR), jnp.int32),
            pltpu.VMEM((2, _HR, D_MODEL), jnp.float32),
            pltpu.SemaphoreType.DMA,
            pltpu.SemaphoreType.DMA,
            pltpu.SemaphoreType.DMA,
            pltpu.SemaphoreType.DMA,
        ],
    )
    def k(eid_hbm, hist_hbm, xf_hbm, gidx_hbm, xs_hbm, counts_hbm,
          eid_v, allhist_v, base_v, counts_v, gidx_v, rows_v,
          sem_a, sem_b, sem_c, sem_d):
        w = _wid()
        tok0 = w * TPW
        in0 = pltpu.async_copy(xf_hbm.at[pl.ds(tok0, _HR)],
                               rows_v.at[0], sem_a)
        in1 = pltpu.async_copy(xf_hbm.at[pl.ds(tok0 + _HR, _HR)],
                               rows_v.at[1], sem_b)
        pltpu.sync_copy(eid_hbm.at[pl.ds(tok0, TPW)], eid_v)
        pltpu.sync_copy(hist_hbm, allhist_v)
        for c in range(E // 16):
            base_v[pl.ds(c * 16, 16)] = jnp.zeros((16,), jnp.int32)

        def acc(wp, carry):
            for c in range(E // 16):
                sl = pl.ds(c * 16, 16)
                base_v[sl] = base_v[sl] + allhist_v[wp, sl]
            return carry

        lax.fori_loop(0, w, acc, 0)

        @pl.when(w == 0)
        def _():
            for c in range(E // 16):
                sl = pl.ds(c * 16, 16)
                tot = jnp.zeros((16,), jnp.int32)
                for wp in range(NW):
                    tot = tot + allhist_v[wp, sl]
                counts_v[sl] = jnp.minimum(tot, CAP)
            pltpu.sync_copy(counts_v, counts_hbm)

        iota = lax.iota(jnp.int32, 16)

        @pl.loop(0, TPW // 16)
        def _(c):
            ev = eid_v[pl.ds(c * 16, 16)]
            gv = jnp.zeros((16,), jnp.int32)
            for lane in range(16):
                e = ev[lane]
                hb = (e // 16) * 16
                bv = base_v[pl.ds(hb, 16)]
                lm = iota == (e - hb)
                r = jnp.max(jnp.where(lm, bv, -1))
                base_v[pl.ds(hb, 16)] = jnp.where(lm, bv + 1, bv)
                g = jnp.where(r < CAP, e * CAP + r, DUMMY)
                gv = jnp.where(iota == lane, g, gv)
            h = c // (_HR // 16)
            col = (c % (_HR // 16)) * 16
            gidx_v[h, pl.ds(col, 16)] = gv

        pltpu.sync_copy(gidx_v, gidx_hbm.at[w])
        in0.wait()
        out0 = pltpu.async_copy(rows_v.at[0], xs_hbm.at[gidx_v.at[0]], sem_c)
        in1.wait()
        out1 = pltpu.async_copy(rows_v.at[1], xs_hbm.at[gidx_v.at[1]], sem_d)
        out0.wait()
        out1.wait()

    return k(eid, hist, xf)


_BM = 64
_MB = CAP // _BM


def _run_mlp(counts, xs, W1, W2, W3):
    def body(counts_ref, xs_ref, w1_ref, w3_ref, w2_ref, out_ref):
        e = pl.program_id(0)

        @pl.when(e == E)
        def _():
            out_ref[...] = jnp.zeros(out_ref.shape, out_ref.dtype)

        @pl.when(e < E)
        def _():
            cur = xs_ref[...]
            curb = cur.astype(jnp.bfloat16)
            dn = (((1,), (1,)), ((), ()))
            h1 = lax.dot_general(curb, w1_ref[0].astype(jnp.bfloat16), dn,
                                 preferred_element_type=jnp.float32)
            h3 = lax.dot_general(curb, w3_ref[0].astype(jnp.bfloat16), dn,
                                 preferred_element_type=jnp.float32)
            hh = h1 * jax.nn.sigmoid(h1) * h3
            o = lax.dot_general(hh.astype(jnp.bfloat16),
                                w2_ref[0].astype(jnp.bfloat16), dn,
                                preferred_element_type=jnp.float32)
            out_ref[0] = o * cur[:, 0:1]

    return pl.pallas_call(
        body,
        grid=(E + 1,),
        in_specs=[
            pl.BlockSpec(memory_space=pltpu.SMEM),
            pl.BlockSpec((CAP, D_MODEL), lambda e: (jnp.minimum(e, E - 1), 0)),
            pl.BlockSpec((1, D_MLP, D_MODEL), lambda e: (jnp.minimum(e, E - 1), 0, 0)),
            pl.BlockSpec((1, D_MLP, D_MODEL), lambda e: (jnp.minimum(e, E - 1), 0, 0)),
            pl.BlockSpec((1, D_MODEL, D_MLP), lambda e: (jnp.minimum(e, E - 1), 0, 0)),
        ],
        out_specs=pl.BlockSpec((1, CAP, D_MODEL), lambda e: (e, 0, 0)),
        out_shape=jax.ShapeDtypeStruct((E + 1, CAP, D_MODEL), jnp.float32),
    )(counts, xs, W1, W3, W2)


def _run_combine(gidx, ys_flat):
    @pl.kernel(
        out_type=jax.ShapeDtypeStruct((N_TOK, D_MODEL), jnp.float32),
        mesh=_mesh(),
        compiler_params=_sc_params(),
        scratch_types=[
            pltpu.VMEM((2, _HR), jnp.int32),
            pltpu.VMEM((2, _HR, D_MODEL), jnp.float32),
            pltpu.SemaphoreType.DMA,
            pltpu.SemaphoreType.DMA,
            pltpu.SemaphoreType.DMA,
            pltpu.SemaphoreType.DMA,
        ],
    )
    def k(gidx_hbm, ys_hbm, out_hbm, idx_v, rows_v, sem_a, sem_b, sem_c, sem_d):
        w = _wid()
        tok0 = w * TPW
        pltpu.sync_copy(gidx_hbm.at[w], idx_v)
        g0 = pltpu.async_copy(ys_hbm.at[idx_v.at[0]], rows_v.at[0], sem_a)
        g1 = pltpu.async_copy(ys_hbm.at[idx_v.at[1]], rows_v.at[1], sem_b)
        g0.wait()
        o0 = pltpu.async_copy(rows_v.at[0], out_hbm.at[pl.ds(tok0, _HR)], sem_c)
        g1.wait()
        o1 = pltpu.async_copy(rows_v.at[1], out_hbm.at[pl.ds(tok0 + _HR, _HR)],
                              sem_d)
        o0.wait()
        o1.wait()

    return k(gidx, ys_flat)


def kernel(x, W_gate, W1, W2, W3):
    batch, pos, d_model = x.shape
    xf = x.reshape(-1, d_model)
    gate_logits = xf @ W_gate.T
    weights = jax.nn.softmax(gate_logits.astype(jnp.float32), axis=1)
    eid = jnp.argmax(weights, axis=1).astype(jnp.int32)

    hist = _run_hist(eid)
    gidx, xs, counts = _run_dispatch(eid, hist, xf)
    ys = _run_mlp(counts, xs, W1, W2, W3)
    out = _run_combine(gidx, ys.reshape((E + 1) * CAP, D_MODEL))
    return out.reshape(batch, pos, d_model)


# speedup vs baseline: 1.0815x; 1.0025x over previous
import jax
import jax.numpy as jnp
from jax import lax
from jax.experimental import pallas as pl
from jax.experimental.pallas import tpu as pltpu
from jax.experimental.pallas import tpu_sc as plsc

E = 64
TOPK = 1
D_MODEL = 768
D_MLP = 1024
CAP = 256
N_TOK = 4096
NC, NS = 2, 16
NW = NC * NS
TPW = N_TOK // NW
DUMMY = E * CAP

def _mesh():
    return plsc.VectorSubcoreMesh(core_axis_name="c", subcore_axis_name="s")


def _sc_params():
    return pltpu.CompilerParams(needs_layout_passes=False)


def _wid():
    return lax.axis_index("s") * NC + lax.axis_index("c")


def _run_hist(eid):
    @pl.kernel(
        out_type=jax.ShapeDtypeStruct((NW, E), jnp.int32),
        mesh=_mesh(),
        compiler_params=_sc_params(),
        scratch_types=[
            pltpu.VMEM((TPW,), jnp.int32),
            pltpu.VMEM((E,), jnp.int32),
        ],
    )
    def k(eid_hbm, hist_hbm, eid_v, hist_v):
        w = _wid()
        iota = lax.iota(jnp.int32, 16)
        pltpu.sync_copy(eid_hbm.at[pl.ds(w * TPW, TPW)], eid_v)
        for c in range(E // 16):
            hist_v[pl.ds(c * 16, 16)] = jnp.zeros((16,), jnp.int32)

        @pl.loop(0, TPW // 16)
        def _(c):
            ev = eid_v[pl.ds(c * 16, 16)]
            for lane in range(16):
                e = ev[lane]
                hb = (e // 16) * 16
                hv = hist_v[pl.ds(hb, 16)]
                lm = iota == (e - hb)
                hist_v[pl.ds(hb, 16)] = jnp.where(lm, hv + 1, hv)

        pltpu.sync_copy(hist_v, hist_hbm.at[w])

    return k(eid)


_HR = TPW // 2


def _run_dispatch(eid, hist, xf):
    @pl.kernel(
        out_type=(
            jax.ShapeDtypeStruct((NW, 2, _HR), jnp.int32),
            jax.ShapeDtypeStruct((DUMMY + 8, D_MODEL), jnp.float32),
        ),
        mesh=_mesh(),
        compiler_params=_sc_params(),
        scratch_types=[
            pltpu.VMEM((TPW,), jnp.int32),
            pltpu.VMEM((NW, E), jnp.int32),
            pltpu.VMEM((E,), jnp.int32),
            pltpu.VMEM((2, _HR), jnp.int32),
            pltpu.VMEM((2, _HR, D_MODEL), jnp.float32),
            pltpu.SemaphoreType.DMA,
            pltpu.SemaphoreType.DMA,
            pltpu.SemaphoreType.DMA,
            pltpu.SemaphoreType.DMA,
        ],
    )
    def k(eid_hbm, hist_hbm, xf_hbm, gidx_hbm, xs_hbm,
          eid_v, allhist_v, base_v, gidx_v, rows_v,
          sem_a, sem_b, sem_c, sem_d):
        w = _wid()
        tok0 = w * TPW
        in0 = pltpu.async_copy(xf_hbm.at[pl.ds(tok0, _HR)],
                               rows_v.at[0], sem_a)
        in1 = pltpu.async_copy(xf_hbm.at[pl.ds(tok0 + _HR, _HR)],
                               rows_v.at[1], sem_b)
        pltpu.sync_copy(eid_hbm.at[pl.ds(tok0, TPW)], eid_v)
        pltpu.sync_copy(hist_hbm, allhist_v)
        for c in range(E // 16):
            base_v[pl.ds(c * 16, 16)] = jnp.zeros((16,), jnp.int32)

        def acc(wp, carry):
            for c in range(E // 16):
                sl = pl.ds(c * 16, 16)
                base_v[sl] = base_v[sl] + allhist_v[wp, sl]
            return carry

        lax.fori_loop(0, w, acc, 0)

        iota = lax.iota(jnp.int32, 16)

        @pl.loop(0, TPW // 16)
        def _(c):
            ev = eid_v[pl.ds(c * 16, 16)]
            gv = jnp.zeros((16,), jnp.int32)
            for lane in range(16):
                e = ev[lane]
                hb = (e // 16) * 16
                bv = base_v[pl.ds(hb, 16)]
                lm = iota == (e - hb)
                r = jnp.max(jnp.where(lm, bv, -1))
                base_v[pl.ds(hb, 16)] = jnp.where(lm, bv + 1, bv)
                g = jnp.where(r < CAP, e * CAP + r, DUMMY)
                gv = jnp.where(iota == lane, g, gv)
            h = c // (_HR // 16)
            col = (c % (_HR // 16)) * 16
            gidx_v[h, pl.ds(col, 16)] = gv

        pltpu.sync_copy(gidx_v, gidx_hbm.at[w])
        in0.wait()
        out0 = pltpu.async_copy(rows_v.at[0], xs_hbm.at[gidx_v.at[0]], sem_c)
        in1.wait()
        out1 = pltpu.async_copy(rows_v.at[1], xs_hbm.at[gidx_v.at[1]], sem_d)
        out0.wait()
        out1.wait()

    return k(eid, hist, xf)


def _run_mlp(xs, W1, W2, W3):
    def body(xs_ref, w1_ref, w3_ref, w2_ref, out_ref):
        e = pl.program_id(0)

        @pl.when(e == E)
        def _():
            out_ref[...] = jnp.zeros(out_ref.shape, out_ref.dtype)

        @pl.when(e < E)
        def _():
            cur = xs_ref[...]
            curb = cur.astype(jnp.bfloat16)
            dn = (((1,), (1,)), ((), ()))
            h1 = lax.dot_general(curb, w1_ref[0].astype(jnp.bfloat16), dn,
                                 preferred_element_type=jnp.float32)
            h3 = lax.dot_general(curb, w3_ref[0].astype(jnp.bfloat16), dn,
                                 preferred_element_type=jnp.float32)
            hh = h1 * jax.nn.sigmoid(h1) * h3
            o = lax.dot_general(hh.astype(jnp.bfloat16),
                                w2_ref[0].astype(jnp.bfloat16), dn,
                                preferred_element_type=jnp.float32)
            out_ref[0] = o * cur[:, 0:1]

    return pl.pallas_call(
        body,
        grid=(E + 1,),
        in_specs=[
            pl.BlockSpec((CAP, D_MODEL), lambda e: (jnp.minimum(e, E - 1), 0)),
            pl.BlockSpec((1, D_MLP, D_MODEL), lambda e: (jnp.minimum(e, E - 1), 0, 0)),
            pl.BlockSpec((1, D_MLP, D_MODEL), lambda e: (jnp.minimum(e, E - 1), 0, 0)),
            pl.BlockSpec((1, D_MODEL, D_MLP), lambda e: (jnp.minimum(e, E - 1), 0, 0)),
        ],
        out_specs=pl.BlockSpec((1, CAP, D_MODEL), lambda e: (e, 0, 0)),
        out_shape=jax.ShapeDtypeStruct((E + 1, CAP, D_MODEL), jnp.float32),
    )(xs, W1, W3, W2)


def _run_combine(gidx, ys_flat):
    @pl.kernel(
        out_type=jax.ShapeDtypeStruct((N_TOK, D_MODEL), jnp.float32),
        mesh=_mesh(),
        compiler_params=_sc_params(),
        scratch_types=[
            pltpu.VMEM((2, _HR), jnp.int32),
            pltpu.VMEM((2, _HR, D_MODEL), jnp.float32),
            pltpu.SemaphoreType.DMA,
            pltpu.SemaphoreType.DMA,
            pltpu.SemaphoreType.DMA,
            pltpu.SemaphoreType.DMA,
        ],
    )
    def k(gidx_hbm, ys_hbm, out_hbm, idx_v, rows_v, sem_a, sem_b, sem_c, sem_d):
        w = _wid()
        tok0 = w * TPW
        pltpu.sync_copy(gidx_hbm.at[w], idx_v)
        g0 = pltpu.async_copy(ys_hbm.at[idx_v.at[0]], rows_v.at[0], sem_a)
        g1 = pltpu.async_copy(ys_hbm.at[idx_v.at[1]], rows_v.at[1], sem_b)
        g0.wait()
        o0 = pltpu.async_copy(rows_v.at[0], out_hbm.at[pl.ds(tok0, _HR)], sem_c)
        g1.wait()
        o1 = pltpu.async_copy(rows_v.at[1], out_hbm.at[pl.ds(tok0 + _HR, _HR)],
                              sem_d)
        o0.wait()
        o1.wait()

    return k(gidx, ys_flat)


def kernel(x, W_gate, W1, W2, W3):
    batch, pos, d_model = x.shape
    xf = x.reshape(-1, d_model)
    gate_logits = xf @ W_gate.T
    weights = jax.nn.softmax(gate_logits.astype(jnp.float32), axis=1)
    eid = jnp.argmax(weights, axis=1).astype(jnp.int32)

    hist = _run_hist(eid)
    gidx, xs = _run_dispatch(eid, hist, xf)
    ys = _run_mlp(xs, W1, W2, W3)
    out = _run_combine(gidx, ys.reshape((E + 1) * CAP, D_MODEL))
    return out.reshape(batch, pos, d_model)
